# baseline (speedup 1.0000x reference)
import jax
import jax.numpy as jnp
from jax import lax
from jax.experimental import pallas as pl
from jax.experimental.pallas import tpu as pltpu
from jax.experimental.pallas import tpu_sc as plsc

B = 1024
K = 100000
L = 16
NW = 32
RPW = B // NW
G = 8
NG = RPW // G
KT = 1408
NCK = 99968 // KT
NVK = KT // L
KMAIN = NCK * KT
KTAIL = K - KMAIN
NPAIR = (NCK - 1) // 2


def _sc_body(s_hbm, v_hbm, o_hbm, vbuf, sbuf, tbuf, obuf, tmat, sems):
    cid = lax.axis_index("c")
    sid = lax.axis_index("s")
    wid = sid * 2 + cid
    base = wid * RPW

    pltpu.sync_copy(v_hbm, vbuf)
    lane = lax.iota(jnp.int32, L)

    for g in range(NG):
        row0 = base + g * G

        def dma(ck, slot):
            return pltpu.make_async_copy(
                s_hbm.at[pl.ds(row0, G), pl.ds(ck * KT, KT)],
                sbuf.at[slot], sems.at[slot])

        def compute_chunk(ck, accs, slot):
            kofs = ck * KT

            def vstep(i, accs):
                off = i * L
                v16 = vbuf[pl.ds(kofs + off, L)]
                return tuple(
                    accs[r] + sbuf[slot, r, pl.ds(off, L)] * v16
                    for r in range(G))

            return lax.fori_loop(0, NVK, vstep, accs)

        dma(0, 0).start()
        dma(1, 1).start()

        def jbody(j, accs):
            for slot in range(2):
                ck = 2 * j + slot
                dma(ck, slot).wait()
                accs = compute_chunk(ck, accs, slot)

                @pl.when(ck + 2 < NCK)
                def _():
                    dma(ck + 2, slot).start()
            return accs

        accs = tuple(jnp.zeros((L,), jnp.float32) for _ in range(G))
        accs = lax.fori_loop(0, NPAIR, jbody, accs)
        dma(NCK - 1, 0).wait()
        accs = compute_chunk(NCK - 1, accs, 0)

        pltpu.make_async_copy(
            s_hbm.at[pl.ds(row0, G), pl.ds(KMAIN, KTAIL)],
            tbuf, sems.at[2]).start()
        pltpu.make_async_copy(
            s_hbm.at[pl.ds(row0, G), pl.ds(KMAIN, KTAIL)],
            tbuf, sems.at[2]).wait()
        for i in range(KTAIL // L):
            v16 = vbuf[pl.ds(KMAIN + i * L, L)]
            accs = tuple(
                accs[r] + tbuf[r, pl.ds(i * L, L)] * v16 for r in range(G))

        for r in range(G):
            tmat[(g % 2) * G + r] = accs[r]

        if g % 2 == 1:
            outv = jnp.zeros((L,), jnp.float32)
            for c in range(L):
                outv = outv + plsc.load_gather(
                    tmat, [lane, jnp.full((L,), c, jnp.int32)])
            obuf[pl.ds((g // 2) * L, L)] = outv

    pltpu.sync_copy(obuf, o_hbm.at[pl.ds(base, RPW)])


def _sc_call(state, values_flat):
    mesh = plsc.VectorSubcoreMesh(core_axis_name="c", subcore_axis_name="s")
    return pl.kernel(
        _sc_body,
        out_type=jax.ShapeDtypeStruct((B,), jnp.float32),
        mesh=mesh,
        compiler_params=pltpu.CompilerParams(needs_layout_passes=False),
        scratch_types=[
            pltpu.VMEM((K,), jnp.float32),
            pltpu.VMEM((2, G, KT), jnp.float32),
            pltpu.VMEM((G, KTAIL), jnp.float32),
            pltpu.VMEM((RPW,), jnp.float32),
            pltpu.VMEM((L, L), jnp.float32),
            pltpu.SemaphoreType.DMA((3,)),
        ],
    )(state, values_flat)


def kernel(state, values):
    out = _sc_call(state, values.reshape(K))
    return out.reshape(B, 1)


# speedup vs baseline: 1.2813x; 1.2813x over previous
import jax
import jax.numpy as jnp
from jax.experimental import pallas as pl
from jax.experimental.pallas import tpu as pltpu

B = 1024
K = 100000
NS = 4
SLAB = B // NS
BB = 16
NSTEP = SLAB // BB


def _body(s0, s1, s2, s3, v_ref, o_ref):
    v = v_ref[...]
    for j, s in enumerate((s0, s1, s2, s3)):
        o_ref[j] = jnp.sum(s[...] * v, axis=1, keepdims=True)


def kernel(state, values):
    values_row = values.reshape(1, K)
    specs = [
        pl.BlockSpec((BB, K), (lambda i, j=j: (NSTEP * j + i, 0)))
        for j in range(NS)
    ]
    out = pl.pallas_call(
        _body,
        grid=(NSTEP,),
        in_specs=specs + [pl.BlockSpec((1, K), lambda i: (0, 0))],
        out_specs=pl.BlockSpec((NS, BB, 1), lambda i: (0, i, 0)),
        out_shape=jax.ShapeDtypeStruct((NS, SLAB, 1), jnp.float32),
        compiler_params=pltpu.CompilerParams(
            dimension_semantics=("parallel",),
        ),
    )(state, state, state, state, values_row)
    return out.reshape(B, 1)
